# baseline (speedup 1.0000x reference)
import functools

import jax
import jax.numpy as jnp
from jax import lax
from jax.experimental import pallas as pl
from jax.experimental.pallas import tpu as pltpu
from jax.experimental.pallas import tpu_sc as plsc

VOCAB = 1000
N = 204800
NC, NS, L = 2, 16, 16
NW = NC * NS
BPW = N // NW
CHUNK = 16
STEPS = BPW // CHUNK
NQ = 4
GROUPS = BPW // L

_NEG = -1e30


def _lse_body(tab_ref, out_ref, copy_ref):
    x = tab_ref[...]
    m = jnp.max(x, axis=1, keepdims=True)
    s = jnp.sum(jnp.exp(x - m), axis=1, keepdims=True)
    lse = m + jnp.log(s)
    out_ref[...] = jax.lax.broadcast_in_dim(lse, (VOCAB, 128), (0, 1))
    copy_ref[...] = x[:, :VOCAB]


def _compute_lse(table_padded):
    return pl.pallas_call(
        _lse_body,
        out_shape=[
            jax.ShapeDtypeStruct((VOCAB, 128), jnp.float32),
            jax.ShapeDtypeStruct((VOCAB, VOCAB), jnp.float32),
        ],
    )(table_padded)


_MESH = plsc.VectorSubcoreMesh(core_axis_name="c", subcore_axis_name="s")


@functools.partial(
    pl.kernel,
    mesh=_MESH,
    compiler_params=pltpu.CompilerParams(
        use_tc_tiling_on_sc=False, needs_layout_passes=False
    ),
    out_type=[
        jax.ShapeDtypeStruct((N, VOCAB), jnp.float32),
        jax.ShapeDtypeStruct((NW, L), jnp.float32),
    ],
    scratch_types=[
        pltpu.VMEM((BPW,), jnp.int32),
        pltpu.VMEM((BPW,), jnp.int32),
        pltpu.VMEM((BPW,), jnp.int32),
        pltpu.VMEM((BPW,), jnp.float32),
        pltpu.VMEM((BPW,), jnp.float32),
        pltpu.VMEM((L,), jnp.float32),
        [pltpu.VMEM((CHUNK, VOCAB), jnp.float32)] * NQ,
        [pltpu.SemaphoreType.DMA] * NQ,
        pltpu.SemaphoreType.DMA,
        pltpu.SemaphoreType.DMA,
    ],
)
def _sc_gather_loss(xb_hbm, yb_hbm, lse_hbm, table_hbm, tabflat_hbm,
                    out_hbm, part_hbm,
                    xb_v, yb_v, fidx_v, gath_v, lsex_v, acc_v,
                    bufs, sems, sem_g, sem_l):
    wid = lax.axis_index("s") * NC + lax.axis_index("c")
    base = wid * BPW
    pltpu.sync_copy(xb_hbm.at[pl.ds(base, BPW)], xb_v)
    pltpu.sync_copy(yb_hbm.at[pl.ds(base, BPW)], yb_v)

    def gather_rows(i, buf, sem):
        return pltpu.async_copy(
            table_hbm.at[xb_v.at[pl.ds(i * CHUNK, CHUNK)]], buf, sem
        )

    for q in range(NQ):
        gather_rows(q, bufs[q], sems[q])

    def fgroup(g, carry):
        fidx_v[pl.ds(g * L, L)] = (
            xb_v[pl.ds(g * L, L)] * VOCAB + yb_v[pl.ds(g * L, L)]
        )
        return carry

    lax.fori_loop(0, GROUPS, fgroup, 0)
    gath_dma = pltpu.async_copy(tabflat_hbm.at[fidx_v], gath_v, sem_g)
    lsex_dma = pltpu.async_copy(lse_hbm.at[xb_v], lsex_v, sem_l)

    def ring(r, carry):
        for q in range(NQ):
            i = r * NQ + q
            pltpu.make_async_copy(
                table_hbm.at[xb_v.at[pl.ds(i * CHUNK, CHUNK)]], bufs[q], sems[q]
            ).wait()
            pltpu.sync_copy(bufs[q], out_hbm.at[pl.ds(base + i * CHUNK, CHUNK)])

            @pl.when(i + NQ < STEPS)
            def _():
                gather_rows(i + NQ, bufs[q], sems[q])
        return carry

    lax.fori_loop(0, STEPS // NQ, ring, 0)

    gath_dma.wait()
    lsex_dma.wait()

    def agroup(g, acc):
        return acc + (lsex_v[pl.ds(g * L, L)] - gath_v[pl.ds(g * L, L)])

    acc = lax.fori_loop(0, GROUPS, agroup, jnp.zeros((L,), jnp.float32))
    acc_v[...] = acc
    pltpu.sync_copy(acc_v, part_hbm.at[wid])


def kernel(xb, yb, table):
    xb_flat = xb.reshape(N).astype(jnp.int32)
    yb_flat = yb.reshape(N).astype(jnp.int32)
    pad = jnp.full((VOCAB, 24), _NEG, dtype=jnp.float32)
    lse2d, tabcopy = _compute_lse(jnp.concatenate([table, pad], axis=1))
    lse = lse2d[:, 0]
    tabflat = tabcopy.reshape(VOCAB * VOCAB)
    logits, parts = _sc_gather_loss(xb_flat, yb_flat, lse, table, tabflat)
    loss = jnp.sum(parts) / jnp.float32(N)
    return (logits, loss)
